# baseline (speedup 1.0000x reference)
import functools
import math

import numpy as np
import jax
import jax.numpy as jnp
from jax import lax
from jax.experimental import pallas as pl
from jax.experimental.pallas import tpu as pltpu
from jax.experimental.pallas import tpu_sc as plsc


def _thresh_sq() -> float:
    t = np.float32(math.pi * 3.0 / 4.0)
    x = np.float32(t * t)
    while np.float32(np.sqrt(x)) > t:
        x = np.nextafter(x, np.float32(0.0))
    while np.float32(np.sqrt(np.nextafter(x, np.float32(np.inf)))) <= t:
        x = np.nextafter(x, np.float32(np.inf))
    return float(x)


_T2 = _thresh_sq()

_NW = 32
_L = 16
_NJ = 32


def _sc_body(z_hbm, out_hbm, buf0, buf1, ov0, ov1, sem0, sem1, osem0, osem1):
    j_w = _JS // _NW
    nstep = j_w // _NJ
    wid = lax.axis_index("s") * 2 + lax.axis_index("c")
    j0 = wid * j_w

    bufs = (buf0, buf1)
    sems = (sem0, sem1)
    ovs = (ov0, ov1)
    osems = (osem0, osem1)

    def start(step):
        src = z_hbm.at[0, pl.ds(j0 + step * _NJ, _NJ), pl.ds(0, 384)]
        return pltpu.async_copy(src, bufs[step % 2], sems[step % 2])

    def out_slice(step):
        out_off = pl.multiple_of((j0 + step * _NJ) * 128, 8)
        return out_hbm.at[pl.ds(out_off, _NJ * 128)]

    start(0)
    for step in range(nstep):
        if step + 1 < nstep:
            start(step + 1)
        buf = bufs[step % 2]
        out_v = ovs[step % 2]
        pltpu.make_async_copy(
            z_hbm.at[0, pl.ds(j0 + step * _NJ, _NJ), pl.ds(0, 384)],
            buf, sems[step % 2]).wait()
        if step >= 2:
            pltpu.make_async_copy(out_v, out_slice(step - 2),
                                  osems[step % 2]).wait()

        @plsc.parallel_loop(0, _NJ, 1, unroll=4)
        def j_body(j, buf=buf, out_v=out_v):
            for u in range(8):
                w0 = buf[j, pl.ds(u * 16, _L)]
                w1 = buf[j, pl.ds(128 + u * 16, _L)]
                w2 = buf[j, pl.ds(256 + u * 16, _L)]
                s = w0 * w0 + w1 * w1 + w2 * w2
                out_v[pl.ds(j * 128 + u * 16, _L)] = (s > _T2).astype(
                    jnp.int32)

        pltpu.async_copy(out_v, out_slice(step), osems[step % 2])
    for step in range(max(nstep - 2, 0), nstep):
        pltpu.make_async_copy(ovs[step % 2], out_slice(step),
                              osems[step % 2]).wait()


_JS = 5120
_NB = 256


def _tc_body(z_ref, o_ref):
    x = z_ref[0]
    w0 = x[:, 0, :]
    w1 = x[:, 1, :]
    w2 = x[:, 2, :]
    s = w0 * w0 + w1 * w1 + w2 * w2
    o_ref[...] = (s > _T2).astype(jnp.int32)


def kernel(t, xi):
    del t
    B = xi.shape[0]
    nj = B // 128
    z4 = xi.reshape(nj, 128, 2, 8)
    z4 = jnp.transpose(z4, (2, 0, 3, 1))
    z = z4.reshape(2, nj, 1024)
    sc_call = functools.partial(
        pl.kernel,
        mesh=plsc.VectorSubcoreMesh(core_axis_name="c", subcore_axis_name="s"),
        compiler_params=pltpu.CompilerParams(
            needs_layout_passes=False, use_tc_tiling_on_sc=False,
            skip_device_barrier=True),
        out_type=jax.ShapeDtypeStruct((_JS * 128,), jnp.int32),
        scratch_types=[
            pltpu.VMEM((_NJ, 384), jnp.float32),
            pltpu.VMEM((_NJ, 384), jnp.float32),
            pltpu.VMEM((_NJ * 128,), jnp.int32),
            pltpu.VMEM((_NJ * 128,), jnp.int32),
            pltpu.SemaphoreType.DMA,
            pltpu.SemaphoreType.DMA,
            pltpu.SemaphoreType.DMA,
            pltpu.SemaphoreType.DMA,
        ],
    )(_sc_body)
    sc_out = sc_call(z)
    n_tc = nj - _JS
    tc_out = pl.pallas_call(
        _tc_body,
        grid=(n_tc // _NB,),
        in_specs=[pl.BlockSpec((1, _NB, 8, 128),
                               lambda j: (0, _JS // _NB + j, 0, 0))],
        out_specs=pl.BlockSpec((_NB, 128), lambda j: (j, 0)),
        out_shape=jax.ShapeDtypeStruct((n_tc, 128), jnp.int32),
    )(z4)
    full = jnp.concatenate([sc_out, tc_out.reshape(-1)])
    return full.astype(jnp.bool_)


# speedup vs baseline: 1.0471x; 1.0471x over previous
import functools
import math

import numpy as np
import jax
import jax.numpy as jnp
from jax import lax
from jax.experimental import pallas as pl
from jax.experimental.pallas import tpu as pltpu
from jax.experimental.pallas import tpu_sc as plsc


def _thresh_sq() -> float:
    t = np.float32(math.pi * 3.0 / 4.0)
    x = np.float32(t * t)
    while np.float32(np.sqrt(x)) > t:
        x = np.nextafter(x, np.float32(0.0))
    while np.float32(np.sqrt(np.nextafter(x, np.float32(np.inf)))) <= t:
        x = np.nextafter(x, np.float32(np.inf))
    return float(x)


_T2 = _thresh_sq()

_NW = 32
_L = 16
_NJ = 64


def _sc_body(z_hbm, out_hbm, buf0, buf1, ov0, ov1, sem0, sem1, osem0, osem1):
    j_w = _JS // _NW
    nstep = j_w // _NJ
    wid = lax.axis_index("s") * 2 + lax.axis_index("c")
    j0 = wid * j_w

    bufs = (buf0, buf1)
    sems = (sem0, sem1)
    ovs = (ov0, ov1)
    osems = (osem0, osem1)

    def start(step):
        src = z_hbm.at[0, pl.ds(j0 + step * _NJ, _NJ), pl.ds(0, 384)]
        return pltpu.async_copy(src, bufs[step % 2], sems[step % 2])

    def out_slice(step):
        out_off = pl.multiple_of((j0 + step * _NJ) * 128, 8)
        return out_hbm.at[pl.ds(out_off, _NJ * 128)]

    start(0)
    for step in range(nstep):
        if step + 1 < nstep:
            start(step + 1)
        buf = bufs[step % 2]
        out_v = ovs[step % 2]
        pltpu.make_async_copy(
            z_hbm.at[0, pl.ds(j0 + step * _NJ, _NJ), pl.ds(0, 384)],
            buf, sems[step % 2]).wait()
        if step >= 2:
            pltpu.make_async_copy(out_v, out_slice(step - 2),
                                  osems[step % 2]).wait()

        @plsc.parallel_loop(0, _NJ, 1, unroll=4)
        def j_body(j, buf=buf, out_v=out_v):
            for u in range(8):
                w0 = buf[j, pl.ds(u * 16, _L)]
                w1 = buf[j, pl.ds(128 + u * 16, _L)]
                w2 = buf[j, pl.ds(256 + u * 16, _L)]
                s = w0 * w0 + w1 * w1 + w2 * w2
                out_v[pl.ds(j * 128 + u * 16, _L)] = (s > _T2).astype(
                    jnp.int32)

        pltpu.async_copy(out_v, out_slice(step), osems[step % 2])
    for step in range(max(nstep - 2, 0), nstep):
        pltpu.make_async_copy(ovs[step % 2], out_slice(step),
                              osems[step % 2]).wait()


_JS = 6144
_NB = 256


def _tc_body(z_ref, o_ref):
    x = z_ref[0]
    w0 = x[:, 0, :]
    w1 = x[:, 1, :]
    w2 = x[:, 2, :]
    s = w0 * w0 + w1 * w1 + w2 * w2
    o_ref[...] = (s > _T2).astype(jnp.int32)


def kernel(t, xi):
    del t
    B = xi.shape[0]
    nj = B // 128
    z4 = xi.reshape(nj, 128, 2, 8)
    z4 = jnp.transpose(z4, (2, 0, 3, 1))
    z = z4.reshape(2, nj, 1024)
    sc_call = functools.partial(
        pl.kernel,
        mesh=plsc.VectorSubcoreMesh(core_axis_name="c", subcore_axis_name="s"),
        compiler_params=pltpu.CompilerParams(
            needs_layout_passes=False, use_tc_tiling_on_sc=False,
            skip_device_barrier=True),
        out_type=jax.ShapeDtypeStruct((_JS * 128,), jnp.int32),
        scratch_types=[
            pltpu.VMEM((_NJ, 384), jnp.float32),
            pltpu.VMEM((_NJ, 384), jnp.float32),
            pltpu.VMEM((_NJ * 128,), jnp.int32),
            pltpu.VMEM((_NJ * 128,), jnp.int32),
            pltpu.SemaphoreType.DMA,
            pltpu.SemaphoreType.DMA,
            pltpu.SemaphoreType.DMA,
            pltpu.SemaphoreType.DMA,
        ],
    )(_sc_body)
    sc_out = sc_call(z)
    n_tc = nj - _JS
    tc_out = pl.pallas_call(
        _tc_body,
        grid=(n_tc // _NB,),
        in_specs=[pl.BlockSpec((1, _NB, 8, 128),
                               lambda j: (0, _JS // _NB + j, 0, 0))],
        out_specs=pl.BlockSpec((_NB, 128), lambda j: (j, 0)),
        out_shape=jax.ShapeDtypeStruct((n_tc, 128), jnp.int32),
    )(z4)
    full = jnp.concatenate([sc_out, tc_out.reshape(-1)])
    return full.astype(jnp.bool_)
